# Initial kernel scaffold; baseline (speedup 1.0000x reference)
import jax
import jax.numpy as jnp
from jax.experimental import pallas as pl


def kernel(world_embed, psi, intent, T):
    raise NotImplementedError("write your pallas kernel here")


# speedup vs baseline: 2.4410x; 2.4410x over previous
import jax
import jax.numpy as jnp
from jax.experimental import pallas as pl
from jax.experimental.pallas import tpu as pltpu

_DEPTH = 8192
_FEAT = 4096
_DECAY = 0.9
_BS = 256


def _body(w_ref, p_ref, i_ref, t_ref, out_ref, carry_ref):
    step = pl.program_id(0)

    @pl.when(step == 0)
    def _init():
        w = jnp.mean(w_ref[...], axis=0)
        p = jnp.mean(p_ref[...], axis=0)
        it = jnp.mean(i_ref[...], axis=0)
        carry_ref[0, :] = jnp.concatenate([w, p, it], axis=-1)

    a = t_ref[...]
    prev = carry_ref[...]
    shifted = jnp.concatenate([prev, a[:-1, :]], axis=0)
    out_ref[...] = a * _DECAY + shifted * (1.0 - _DECAY)
    carry_ref[...] = a[-1:, :]


def kernel(world_embed, psi, intent, T):
    grid = (_DEPTH // _BS,)
    return pl.pallas_call(
        _body,
        grid=grid,
        in_specs=[
            pl.BlockSpec(world_embed.shape, lambda i: (0, 0)),
            pl.BlockSpec(psi.shape, lambda i: (0, 0)),
            pl.BlockSpec(intent.shape, lambda i: (0, 0)),
            pl.BlockSpec((_BS, _FEAT), lambda i: (i, 0)),
        ],
        out_specs=pl.BlockSpec((_BS, _FEAT), lambda i: (i, 0)),
        out_shape=jax.ShapeDtypeStruct((_DEPTH, _FEAT), jnp.float32),
        scratch_shapes=[pltpu.VMEM((1, _FEAT), jnp.float32)],
    )(world_embed, psi, intent, T)
